# baseline (speedup 1.0000x reference)
import dataclasses
import functools

import jax
import jax.numpy as jnp
from jax import lax
from jax.experimental import pallas as pl
from jax.experimental.pallas import tpu as pltpu
from jax.experimental.pallas import tpu_sc as plsc

_DIM = 32
_W = 200

try:
    _info = plsc.get_sparse_core_info()
    _NUM_CORES, _NUM_SUBCORES = _info.num_cores, _info.num_subcores
except Exception:
    _NUM_CORES, _NUM_SUBCORES = 2, 16


def kernel(x, table):
    batch, length = x.shape
    n = batch * length
    vocab, dim = table.shape
    idx = x.reshape(n)
    table128 = table.reshape(vocab // 4, 128)

    workers = _NUM_CORES * _NUM_SUBCORES
    per_worker = n // workers
    n_chunks = per_worker // _W
    assert per_worker % _W == 0 and n_chunks % 2 == 0 and _W % length == 0

    mesh = plsc.VectorSubcoreMesh(core_axis_name="c", subcore_axis_name="s")

    cp = pltpu.CompilerParams()
    if "needs_layout_passes" in pltpu.CompilerParams.__dataclass_fields__:
        cp = dataclasses.replace(cp, needs_layout_passes=False)

    @functools.partial(
        pl.kernel,
        mesh=mesh,
        compiler_params=cp,
        out_type=jax.ShapeDtypeStruct((batch, length, dim), table.dtype),
        scratch_types=[
            pltpu.VMEM((_W,), jnp.int32),
            pltpu.VMEM((_W,), jnp.int32),
            pltpu.VMEM((_W,), jnp.int32),
            pltpu.VMEM((_W,), jnp.int32),
            pltpu.VMEM((_W,), jnp.int32),
            pltpu.VMEM((_W,), jnp.int32),
            pltpu.VMEM((_W, 128), jnp.float32),
            pltpu.VMEM((_W, 128), jnp.float32),
            pltpu.VMEM((_W, _DIM), jnp.float32),
            pltpu.VMEM((_W, _DIM), jnp.float32),
            pltpu.SemaphoreType.DMA,
            pltpu.SemaphoreType.DMA,
            pltpu.SemaphoreType.DMA,
            pltpu.SemaphoreType.DMA,
            pltpu.SemaphoreType.DMA,
            pltpu.SemaphoreType.DMA,
        ],
    )
    def gather_kernel(table_hbm, idx_hbm, out_hbm,
                      xi_a, xi_b, hi_a, hi_b, co_a, co_b,
                      buf_a, buf_b, out_a, out_b,
                      xs_a, xs_b, gs_a, gs_b, os_a, os_b):
        wid = lax.axis_index("s") * _NUM_CORES + lax.axis_index("c")
        base = wid * per_worker
        iota16 = lax.iota(jnp.int32, 16)

        X = (xi_a, xi_b)
        HI = (hi_a, hi_b)
        CO = (co_a, co_b)
        BUF = (buf_a, buf_b)
        OUT = (out_a, out_b)
        XS = (xs_a, xs_b)
        GS = (gs_a, gs_b)
        OS = (os_a, os_b)

        def x_copy(c, p):
            return pltpu.make_async_copy(
                idx_hbm.at[pl.ds(base + c * _W, _W)], X[p], XS[p])

        def g_copy(p):
            return pltpu.make_async_copy(table_hbm.at[HI[p]], BUF[p], GS[p])

        n_b = _W // length

        def o_copies(c, p):
            cb = (base + c * _W) // length
            return [
                pltpu.make_async_copy(
                    OUT[p].at[pl.ds(k * length, length)],
                    out_hbm.at[cb + k], OS[p])
                for k in range(n_b)
            ]

        def o_start(c, p):
            for cp_ in o_copies(c, p):
                cp_.start()

        def o_wait(c, p):
            for cp_ in o_copies(c, p):
                cp_.wait()

        n_full = _W // 16

        def compute_hico(p):
            @pl.loop(0, n_full)
            def _(g):
                sl = pl.ds(g * 16, 16)
                v = X[p][sl]
                HI[p][sl] = lax.shift_right_logical(v, 2)
                CO[p][sl] = lax.shift_left(jnp.bitwise_and(v, 3), 5)
            if _W % 16:
                sl = pl.ds(_W - 16, 16)
                v = X[p][sl]
                HI[p][sl] = lax.shift_right_logical(v, 2)
                CO[p][sl] = lax.shift_left(jnp.bitwise_and(v, 3), 5)

        def select_group(p, r0):
            rows16 = r0 + iota16
            co16 = CO[p][pl.ds(r0, 16)]
            zeros16 = rows16 * 0
            for j in range(_DIM):
                vals = plsc.load_gather(BUF[p], [rows16, co16 + j])
                plsc.store_scatter(OUT[p], [rows16, zeros16 + j], vals)

        def select(p):
            @pl.loop(0, n_full)
            def _(g):
                select_group(p, g * 16)
            if _W % 16:
                select_group(p, _W - 16)

        def step(c, p, i, first_pair, near_end):
            q = 1 - p
            x_copy(c, p).wait()
            compute_hico(p)
            g_copy(p).start()
            if near_end is None:
                x_copy(c + 1, q).start()
            else:
                @pl.when(i < n_chunks // 2 - 1)
                def _():
                    x_copy(c + 1, q).start()

            def tail():
                g_copy(q).wait()
                @pl.when(i >= (2 if first_pair is not None else 1))
                def _():
                    o_wait(c - 3, q)
                select(q)
                o_start(c - 1, q)

            if first_pair is not None:
                @pl.when(i >= 1)
                def _():
                    tail()
            else:
                tail()

        x_copy(0, 0).start()

        @pl.loop(0, n_chunks // 2)
        def _(i):
            c_even = i * 2
            step(c_even, 0, i, first_pair=True, near_end=None)
            step(c_even + 1, 1, i, first_pair=None, near_end=True)

        last = n_chunks - 1
        p_last = last % 2
        g_copy(p_last).wait()
        o_wait(last - 2, p_last)
        select(p_last)
        o_start(last, p_last)
        o_wait(last - 1, 1 - p_last)
        o_wait(last, p_last)

    return gather_kernel(table128, idx)


# speedup vs baseline: 1.0003x; 1.0003x over previous
import dataclasses
import functools

import jax
import jax.numpy as jnp
from jax import lax
from jax.experimental import pallas as pl
from jax.experimental.pallas import tpu as pltpu
from jax.experimental.pallas import tpu_sc as plsc

_DIM = 32
_W = 200

try:
    _info = plsc.get_sparse_core_info()
    _NUM_CORES, _NUM_SUBCORES = _info.num_cores, _info.num_subcores
except Exception:
    _NUM_CORES, _NUM_SUBCORES = 2, 16


def kernel(x, table):
    batch, length = x.shape
    n = batch * length
    vocab, dim = table.shape
    table128 = table.reshape(vocab // 4, 128)

    workers = _NUM_CORES * _NUM_SUBCORES
    per_worker = n // workers
    n_chunks = per_worker // _W
    assert per_worker % _W == 0 and n_chunks % 2 == 0 and _W % length == 0

    mesh = plsc.VectorSubcoreMesh(core_axis_name="c", subcore_axis_name="s")

    cp = pltpu.CompilerParams()
    if "needs_layout_passes" in pltpu.CompilerParams.__dataclass_fields__:
        cp = dataclasses.replace(cp, needs_layout_passes=False)

    @functools.partial(
        pl.kernel,
        mesh=mesh,
        compiler_params=cp,
        out_type=jax.ShapeDtypeStruct((batch, length, dim), table.dtype),
        scratch_types=[
            pltpu.VMEM((8, 50), jnp.int32),
            pltpu.VMEM((8, 50), jnp.int32),
            pltpu.VMEM((_W,), jnp.int32),
            pltpu.VMEM((_W,), jnp.int32),
            pltpu.VMEM((_W,), jnp.int32),
            pltpu.VMEM((_W,), jnp.int32),
            pltpu.VMEM((_W, 128), jnp.float32),
            pltpu.VMEM((_W, 128), jnp.float32),
            pltpu.VMEM((_W, _DIM), jnp.float32),
            pltpu.VMEM((_W, _DIM), jnp.float32),
            pltpu.SemaphoreType.DMA,
            pltpu.SemaphoreType.DMA,
            pltpu.SemaphoreType.DMA,
            pltpu.SemaphoreType.DMA,
            pltpu.SemaphoreType.DMA,
            pltpu.SemaphoreType.DMA,
        ],
    )
    def gather_kernel(table_hbm, idx_hbm, out_hbm,
                      xi_a, xi_b, hi_a, hi_b, co_a, co_b,
                      buf_a, buf_b, out_a, out_b,
                      xs_a, xs_b, gs_a, gs_b, os_a, os_b):
        wid = lax.axis_index("s") * _NUM_CORES + lax.axis_index("c")
        base = wid * per_worker
        iota16 = lax.iota(jnp.int32, 16)

        X = (xi_a, xi_b)
        HI = (hi_a, hi_b)
        CO = (co_a, co_b)
        BUF = (buf_a, buf_b)
        OUT = (out_a, out_b)
        XS = (xs_a, xs_b)
        GS = (gs_a, gs_b)
        OS = (os_a, os_b)

        n_b = _W // length
        base_b = base // length

        def x_copy(slab, p):
            off = pl.multiple_of(base_b + slab * 2 * n_b, 2 * n_b)
            return pltpu.make_async_copy(
                idx_hbm.at[pl.ds(off, 2 * n_b)], X[p], XS[p])

        def g_copy(p):
            return pltpu.make_async_copy(table_hbm.at[HI[p]], BUF[p], GS[p])

        def o_copies(c, p):
            cb = (base + c * _W) // length
            return [
                pltpu.make_async_copy(
                    OUT[p].at[pl.ds(k * length, length)],
                    out_hbm.at[cb + k], OS[p])
                for k in range(n_b)
            ]

        def o_start(c, p):
            for cp_ in o_copies(c, p):
                cp_.start()

        def o_wait(c, p):
            for cp_ in o_copies(c, p):
                cp_.wait()

        n_full = _W // 16

        row_windows = [0, 16, 32, length - 16]

        def compute_hico(p, r0):
            for r in range(n_b):
                for w in row_windows:
                    v = X[p][r0 + r, pl.ds(w, 16)]
                    sl = pl.ds(r * length + w, 16)
                    HI[p][sl] = lax.shift_right_logical(v, 2)
                    CO[p][sl] = lax.shift_left(jnp.bitwise_and(v, 3), 5)

        def select_group(p, r0):
            rows16 = r0 + iota16
            co16 = CO[p][pl.ds(r0, 16)]
            zeros16 = rows16 * 0
            for j in range(_DIM):
                vals = plsc.load_gather(BUF[p], [rows16, co16 + j])
                plsc.store_scatter(OUT[p], [rows16, zeros16 + j], vals)

        def select(p):
            @pl.loop(0, n_full)
            def _(g):
                select_group(p, g * 16)
            if _W % 16:
                select_group(p, _W - 16)

        def step(c, p, i, first_pair, near_end):
            q = 1 - p
            x_copy(i, p).wait()
            compute_hico(p, 0 if p == 0 else n_b)
            g_copy(p).start()
            if near_end is None:
                x_copy(i, q).start()
            else:
                @pl.when(i < n_chunks // 2 - 1)
                def _():
                    x_copy(i + 1, q).start()

            def tail():
                g_copy(q).wait()
                @pl.when(i >= (2 if first_pair is not None else 1))
                def _():
                    o_wait(c - 3, q)
                select(q)
                o_start(c - 1, q)

            if first_pair is not None:
                @pl.when(i >= 1)
                def _():
                    tail()
            else:
                tail()

        x_copy(0, 0).start()

        @pl.loop(0, n_chunks // 2)
        def _(i):
            c_even = i * 2
            step(c_even, 0, i, first_pair=True, near_end=None)
            step(c_even + 1, 1, i, first_pair=None, near_end=True)

        last = n_chunks - 1
        p_last = last % 2
        g_copy(p_last).wait()
        o_wait(last - 2, p_last)
        select(p_last)
        o_start(last, p_last)
        o_wait(last - 1, 1 - p_last)
        o_wait(last, p_last)

    return gather_kernel(table128, x)


# speedup vs baseline: 1.7328x; 1.7323x over previous
import dataclasses
import functools

import jax
import jax.numpy as jnp
from jax import lax
from jax.experimental import pallas as pl
from jax.experimental.pallas import tpu as pltpu
from jax.experimental.pallas import tpu_sc as plsc

_DIM = 32
_W = 400

try:
    _info = plsc.get_sparse_core_info()
    _NUM_CORES, _NUM_SUBCORES = _info.num_cores, _info.num_subcores
except Exception:
    _NUM_CORES, _NUM_SUBCORES = 2, 16


def kernel(x, table):
    batch, length = x.shape
    n = batch * length
    vocab, dim = table.shape

    workers = _NUM_CORES * _NUM_SUBCORES
    per_worker = n // workers
    n_chunks = per_worker // _W
    n_b = _W // length
    assert per_worker % _W == 0 and n_chunks % 2 == 0 and _W % length == 0

    mesh = plsc.VectorSubcoreMesh(core_axis_name="c", subcore_axis_name="s")

    cp = pltpu.CompilerParams()
    fields = pltpu.CompilerParams.__dataclass_fields__
    if "needs_layout_passes" in fields:
        cp = dataclasses.replace(cp, needs_layout_passes=False)
    if "use_tc_tiling_on_sc" in fields:
        cp = dataclasses.replace(cp, use_tc_tiling_on_sc=False)

    @functools.partial(
        pl.kernel,
        mesh=mesh,
        compiler_params=cp,
        out_type=jax.ShapeDtypeStruct((batch, length, dim), table.dtype),
        scratch_types=[
            pltpu.VMEM((n_b, length), jnp.int32),
            pltpu.VMEM((n_b, length), jnp.int32),
            pltpu.VMEM((_W,), jnp.int32),
            pltpu.VMEM((_W,), jnp.int32),
            pltpu.VMEM((_W, _DIM), jnp.float32),
            pltpu.VMEM((_W, _DIM), jnp.float32),
            pltpu.SemaphoreType.DMA,
            pltpu.SemaphoreType.DMA,
            pltpu.SemaphoreType.DMA,
            pltpu.SemaphoreType.DMA,
            pltpu.SemaphoreType.DMA,
            pltpu.SemaphoreType.DMA,
        ],
    )
    def gather_kernel(table_hbm, idx_hbm, out_hbm,
                      xi_a, xi_b, hi_a, hi_b, ob_a, ob_b,
                      xs_a, xs_b, gs_a, gs_b, os_a, os_b):
        wid = lax.axis_index("s") * _NUM_CORES + lax.axis_index("c")
        base_b = wid * (per_worker // length)

        X = (xi_a, xi_b)
        HI = (hi_a, hi_b)
        OB = (ob_a, ob_b)
        XS = (xs_a, xs_b)
        GS = (gs_a, gs_b)
        OS = (os_a, os_b)

        def x_copy(c, p):
            off = pl.multiple_of(base_b + c * n_b, n_b)
            return pltpu.make_async_copy(
                idx_hbm.at[pl.ds(off, n_b)], X[p], XS[p])

        def g_copy(p):
            return pltpu.make_async_copy(table_hbm.at[HI[p]], OB[p], GS[p])

        def o_copies(c, p):
            cb = base_b + c * n_b
            return [
                pltpu.make_async_copy(
                    OB[p].at[pl.ds(k * length, length)],
                    out_hbm.at[cb + k], OS[p])
                for k in range(n_b)
            ]

        def o_start(c, p):
            for cp_ in o_copies(c, p):
                cp_.start()

        def o_wait(c, p):
            for cp_ in o_copies(c, p):
                cp_.wait()

        row_windows = [0, 16, 32, length - 16]

        def flatten_idx(p):
            for r in range(n_b):
                for w in row_windows:
                    HI[p][pl.ds(r * length + w, 16)] = X[p][r, pl.ds(w, 16)]

        def step(c, p, i, first_pair, near_end):
            q = 1 - p
            x_copy(c, p).wait()
            flatten_idx(p)

            @pl.when(i >= 1)
            def _():
                o_wait(c - 2, p)

            g_copy(p).start()
            if near_end is None:
                x_copy(c + 1, q).start()
            else:
                @pl.when(i < n_chunks // 2 - 1)
                def _():
                    x_copy(c + 1, q).start()

            def tail():
                g_copy(q).wait()
                o_start(c - 1, q)

            if first_pair is not None:
                @pl.when(i >= 1)
                def _():
                    tail()
            else:
                tail()

        x_copy(0, 0).start()

        @pl.loop(0, n_chunks // 2)
        def _(i):
            c_even = i * 2
            step(c_even, 0, i, first_pair=True, near_end=None)
            step(c_even + 1, 1, i, first_pair=None, near_end=True)

        last = n_chunks - 1
        p_last = last % 2
        g_copy(p_last).wait()
        o_start(last, p_last)
        o_wait(last - 1, 1 - p_last)
        o_wait(last, p_last)

    return gather_kernel(table, x)
